# baseline (speedup 1.0000x reference)
import dataclasses

import jax
import jax.numpy as jnp
from jax import lax
from jax.experimental import pallas as pl
from jax.experimental.pallas import tpu as pltpu
from jax.experimental.pallas import tpu_sc as plsc

_N = 8732
_NP = 8736
_M = 50
_MP = 64
_IOU_THRESH = 0.5
_STDS = (0.1, 0.1, 0.2, 0.2)
_BIG = 2**30


def _iou_parts(at, gt):
    cx, cy, w, h = at[0:1, :], at[1:2, :], at[2:3, :], at[3:4, :]
    ax1 = cx - w * 0.5
    ay1 = cy - h * 0.5
    ax2 = cx + w * 0.5
    ay2 = cy + h * 0.5
    gx1, gy1, gx2, gy2 = gt[:, 0:1], gt[:, 1:2], gt[:, 2:3], gt[:, 3:4]
    iw = jnp.maximum(jnp.minimum(ax2, gx2) - jnp.maximum(ax1, gx1), 0.0)
    ih = jnp.maximum(jnp.minimum(ay2, gy2) - jnp.maximum(ay1, gy1), 0.0)
    inter = iw * ih
    area_a = (ax2 - ax1) * (ay2 - ay1)
    area_g = (gx2 - gx1) * (gy2 - gy1)
    iou = inter / (area_a + area_g - inter + 1e-12)
    return iou, (ax1, ay1, ax2, ay2)


def _tc_a_body(a_ref, gt_ref, at_ref, iou_ref, ba_ref, mm_ref):
    at = jnp.transpose(a_ref[...])
    at_ref[...] = at
    iou, _ = _iou_parts(at, gt_ref[...])
    iou_ref[:, :_N] = iou
    iou_ref[:, _N:] = jnp.full((_M, _NP - _N), -1.0, jnp.float32)
    a_iota = lax.broadcasted_iota(jnp.int32, (_M, _N), 1)
    g_iota = lax.broadcasted_iota(jnp.int32, (_M, _N), 0)
    bmax = jnp.max(iou, axis=1, keepdims=True)
    amin = jnp.min(
        jnp.where(iou == bmax, a_iota, jnp.int32(_BIG)), axis=1, keepdims=True
    )
    neg2 = jnp.full((_MP - _M, 1), -1073741824, jnp.int32)
    ba_ref[...] = jnp.concatenate(
        [
            lax.bitcast_convert_type(bmax, jnp.int32),
            neg2,
            amin,
            jnp.zeros((_MP - _M, 1), jnp.int32),
        ],
        axis=0,
    )
    mm_max = jnp.max(iou, axis=0, keepdims=True)
    mm_arg = jnp.min(
        jnp.where(iou == mm_max, g_iota, jnp.int32(_BIG)), axis=0, keepdims=True
    )
    mm_ref[...] = jnp.where(mm_max >= _IOU_THRESH, mm_arg.astype(jnp.float32), -1.0)


def _sc_b_body(iou_hbm, ba_hbm, out_hbm,
               ba_v, b_v, a_v, mr_v, pen_v, row_v, acc_v, acci_v, sem):
    is0 = (lax.axis_index("c") == 0) & (lax.axis_index("s") == 0)

    @pl.when(is0)
    def _():
        pltpu.sync_copy(ba_hbm, ba_v)
        lanes = lax.broadcasted_iota(jnp.int32, (16,), 0)

        for k in range(_MP // 16):
            b_v[pl.ds(k * 16, 16)] = lax.bitcast_convert_type(
                ba_v[pl.ds(k * 16, 16)], jnp.float32
            )
            a_v[pl.ds(k * 16, 16)] = ba_v[pl.ds(_MP + k * 16, 16)]
            mr_v[pl.ds(k * 16, 16)] = jnp.full((16,), -1, jnp.int32)

        zeros16 = jnp.zeros((16,), jnp.float32)

        @pl.loop(0, _NP // (16 * 14))
        def _(i):
            for u in range(14):
                pen_v[pl.ds((i * 14 + u) * 16, 16)] = zeros16

        def scan_row_best():
            acc_v[...] = row_v[pl.ds(0, 16)] + pen_v[pl.ds(0, 16)]
            acci_v[...] = lanes

            @pl.loop(0, (_NP // 16 - 1) // 5)
            def _(i):
                for u in range(5):
                    j = 1 + i * 5 + u
                    v = row_v[pl.ds(j * 16, 16)] + pen_v[pl.ds(j * 16, 16)]
                    cur = acc_v[...]
                    take = v > cur
                    acc_v[...] = jnp.where(take, v, cur)
                    acci_v[...] = jnp.where(take, lanes + j * 16, acci_v[...])

            mx = jnp.max(acc_v[...])
            arg = jnp.min(jnp.where(acc_v[...] == mx, acci_v[...], jnp.int32(_BIG)))
            return mx, arg

        def update_gt(g, newb, newa, alive_only):
            off = (g // 16) * 16
            lsel = lanes == (g % 16)
            if alive_only:
                lsel = lsel & (b_v[pl.ds(off, 16)] > -1.5)
            b_v[pl.ds(off, 16)] = jnp.where(lsel, newb, b_v[pl.ds(off, 16)])
            a_v[pl.ds(off, 16)] = jnp.where(lsel, newa, a_v[pl.ds(off, 16)])

        def rescan(c):
            pltpu.async_copy(iou_hbm.at[c], row_v, sem).wait()
            mx2, arg2 = scan_row_best()
            update_gt(c, mx2, arg2, False)

        def select():
            val = b_v[pl.ds(0, 16)]
            pk = a_v[pl.ds(0, 16)] * _MP + lanes
            for k in range(1, _MP // 16):
                v = b_v[pl.ds(k * 16, 16)]
                p = a_v[pl.ds(k * 16, 16)] * _MP + (lanes + k * 16)
                take = (v > val) | ((v == val) & (p < pk))
                val = jnp.where(take, v, val)
                pk = jnp.where(take, p, pk)
            mx = jnp.max(val)
            pkm = jnp.min(jnp.where(val == mx, pk, jnp.int32(_BIG)))
            r = pkm // _MP
            c = pkm % _MP
            po = (r // 16) * 16
            pr = jnp.max(jnp.where(lanes == (r % 16), pen_v[pl.ds(po, 16)], -1e30))
            return r, c, mx > 1e-12, pr < -2.5

        def commit(r, c):
            goff = (c // 16) * 16
            gsel = lanes == (c % 16)
            mr_v[pl.ds(goff, 16)] = jnp.where(gsel, r, mr_v[pl.ds(goff, 16)])
            b_v[pl.ds(goff, 16)] = jnp.where(
                gsel, jnp.float32(-2.0), b_v[pl.ds(goff, 16)]
            )
            po = (r // 16) * 16
            psel = lanes == (r % 16)
            pen_v[pl.ds(po, 16)] = jnp.where(
                psel, jnp.float32(-3.0), pen_v[pl.ds(po, 16)]
            )

        @pl.loop(0, _M)
        def _(_round):
            r1, c1, valid1, stale1 = select()

            @pl.when(valid1 & jnp.logical_not(stale1))
            def _():
                commit(r1, c1)

            @pl.when(valid1 & stale1)
            def _():
                rescan(c1)
                r2, c2, valid2, stale2 = select()

                @pl.when(valid2 & jnp.logical_not(stale2))
                def _():
                    commit(r2, c2)

                @pl.when(valid2 & stale2)
                def _():
                    rescan(c2)
                    r3, c3, valid3, stale3 = select()

                    @pl.when(valid3 & jnp.logical_not(stale3))
                    def _():
                        commit(r3, c3)

                    @pl.when(valid3 & stale3)
                    def _():
                        @pl.loop(0, _M)
                        def _(g):
                            pltpu.async_copy(iou_hbm.at[g], row_v, sem).wait()
                            mxg, argg = scan_row_best()
                            update_gt(g, mxg, argg, True)

                        r4, c4, valid4, _stale4 = select()

                        @pl.when(valid4)
                        def _():
                            commit(r4, c4)

        pltpu.sync_copy(mr_v, out_hbm)


def _tc_c_body(at_ref, gt_ref, gid_ref, mr_ref, mm_ref,
               cls_ref, box_ref, msk_ref):
    at = at_ref[...]
    gt = gt_ref[...]
    _, (ax1, ay1, ax2, ay2) = _iou_parts(at, gt)
    gx1, gy1, gx2, gy2 = gt[:, 0:1], gt[:, 1:2], gt[:, 2:3], gt[:, 3:4]

    arow = lax.broadcasted_iota(jnp.int32, (1, _N), 1)
    g_iota64 = lax.broadcasted_iota(jnp.int32, (_MP, _N), 0)
    mr = mr_ref[...]
    bip = jnp.max(
        jnp.where(mr == arow, g_iota64, jnp.int32(-1)), axis=0, keepdims=True
    )

    mm = mm_ref[...]
    matches = jnp.where(bip >= 0, bip.astype(jnp.float32), mm)
    pos = matches >= 0.0
    safe = jnp.clip(matches, 0.0, float(_M - 1)).astype(jnp.int32)

    g_iota = lax.broadcasted_iota(jnp.int32, (_M, _N), 0)
    onehot = g_iota == safe

    def gsel(col):
        return jnp.max(jnp.where(onehot, col, -1e30), axis=0, keepdims=True)

    gid = gid_ref[...]
    rid = gsel(gid)
    rx1 = gsel(gx1)
    ry1 = gsel(gy1)
    rx2 = gsel(gx2)
    ry2 = gsel(gy2)

    cls_ref[...] = jnp.where(pos, rid + 1.0, 0.0)

    gw = rx2 - rx1
    gh = ry2 - ry1
    gx = rx1 + gw * 0.5
    gy = ry1 + gh * 0.5
    aw = ax2 - ax1
    ah = ay2 - ay1
    axc = ax1 + aw * 0.5
    ayc = ay1 + ah * 0.5
    t0 = ((gx - axc) / (aw + 1e-12)) / _STDS[0]
    t1 = ((gy - ayc) / (ah + 1e-12)) / _STDS[1]
    t2 = jnp.log(jnp.maximum(gw / (aw + 1e-12), 1e-12)) / _STDS[2]
    t3 = jnp.log(jnp.maximum(gh / (ah + 1e-12), 1e-12)) / _STDS[3]
    codes = jnp.concatenate([t0, t1, t2, t3], axis=0)

    posf = pos.astype(jnp.float32)
    box_ref[...] = jnp.transpose(codes * posf)
    msk_ref[...] = jnp.transpose(jnp.broadcast_to(posf, (4, _N)))


def _run_sc_match(iou_p, ba):
    mesh = plsc.VectorSubcoreMesh(
        core_axis_name="c", subcore_axis_name="s", num_cores=2, num_subcores=16
    )
    cp = pltpu.CompilerParams()
    if "needs_layout_passes" in pltpu.CompilerParams.__dataclass_fields__:
        cp = dataclasses.replace(cp, needs_layout_passes=False)
    return pl.kernel(
        _sc_b_body,
        out_type=jax.ShapeDtypeStruct((_MP,), jnp.int32),
        mesh=mesh,
        scratch_types=[
            pltpu.VMEM((2 * _MP,), jnp.int32),
            pltpu.VMEM((_MP,), jnp.float32),
            pltpu.VMEM((_MP,), jnp.int32),
            pltpu.VMEM((_MP,), jnp.int32),
            pltpu.VMEM((_NP,), jnp.float32),
            pltpu.VMEM((_NP,), jnp.float32),
            pltpu.VMEM((16,), jnp.float32),
            pltpu.VMEM((16,), jnp.int32),
            pltpu.SemaphoreType.DMA,
        ],
        compiler_params=cp,
    )(iou_p, ba)


@jax.jit
def kernel(anchors, gt_boxes, gt_ids):
    anchors_t, iou_p, ba, mm = pl.pallas_call(
        _tc_a_body,
        out_shape=(
            jax.ShapeDtypeStruct((4, _N), jnp.float32),
            jax.ShapeDtypeStruct((_M, _NP), jnp.float32),
            jax.ShapeDtypeStruct((2 * _MP, 1), jnp.int32),
            jax.ShapeDtypeStruct((1, _N), jnp.float32),
        ),
    )(anchors, gt_boxes)

    mr = _run_sc_match(iou_p, ba.reshape(2 * _MP))

    cls, box, msk = pl.pallas_call(
        _tc_c_body,
        out_shape=(
            jax.ShapeDtypeStruct((1, _N), jnp.float32),
            jax.ShapeDtypeStruct((_N, 4), jnp.float32),
            jax.ShapeDtypeStruct((_N, 4), jnp.float32),
        ),
    )(anchors_t, gt_boxes, gt_ids, mr.reshape(_MP, 1), mm)

    return cls, box[None, :, :], msk[None, :, :]


# speedup vs baseline: 1.0940x; 1.0940x over previous
import dataclasses

import jax
import jax.numpy as jnp
from jax import lax
from jax.experimental import pallas as pl
from jax.experimental.pallas import tpu as pltpu
from jax.experimental.pallas import tpu_sc as plsc

_N = 8732
_NP = 8736
_M = 50
_MP = 64
_IOU_THRESH = 0.5
_STDS = (0.1, 0.1, 0.2, 0.2)
_BIG = 2**30


def _iou_parts(at, gt):
    cx, cy, w, h = at[0:1, :], at[1:2, :], at[2:3, :], at[3:4, :]
    ax1 = cx - w * 0.5
    ay1 = cy - h * 0.5
    ax2 = cx + w * 0.5
    ay2 = cy + h * 0.5
    gx1, gy1, gx2, gy2 = gt[:, 0:1], gt[:, 1:2], gt[:, 2:3], gt[:, 3:4]
    iw = jnp.maximum(jnp.minimum(ax2, gx2) - jnp.maximum(ax1, gx1), 0.0)
    ih = jnp.maximum(jnp.minimum(ay2, gy2) - jnp.maximum(ay1, gy1), 0.0)
    inter = iw * ih
    area_a = (ax2 - ax1) * (ay2 - ay1)
    area_g = (gx2 - gx1) * (gy2 - gy1)
    iou = inter / (area_a + area_g - inter + 1e-12)
    return iou, (ax1, ay1, ax2, ay2)


def _tc_a_body(at_ref, gt_ref, iou_ref, ba_ref, mm_ref):
    iou, _ = _iou_parts(at_ref[...], gt_ref[...])
    iou_ref[:, :_N] = iou
    iou_ref[:, _N:] = jnp.full((_M, _NP - _N), -1.0, jnp.float32)
    a_iota = lax.broadcasted_iota(jnp.int32, (_M, _N), 1)
    g_iota = lax.broadcasted_iota(jnp.int32, (_M, _N), 0)
    bmax = jnp.max(iou, axis=1, keepdims=True)
    amin = jnp.min(
        jnp.where(iou == bmax, a_iota, jnp.int32(_BIG)), axis=1, keepdims=True
    )
    neg2 = jnp.full((_MP - _M, 1), -1073741824, jnp.int32)
    ba_ref[...] = jnp.concatenate(
        [
            lax.bitcast_convert_type(bmax, jnp.int32),
            neg2,
            amin,
            jnp.zeros((_MP - _M, 1), jnp.int32),
        ],
        axis=0,
    )
    mm_max = jnp.max(iou, axis=0, keepdims=True)
    mm_arg = jnp.min(
        jnp.where(iou == mm_max, g_iota, jnp.int32(_BIG)), axis=0, keepdims=True
    )
    mm_ref[...] = jnp.where(mm_max >= _IOU_THRESH, mm_arg.astype(jnp.float32), -1.0)


def _sc_b_body(iou_hbm, ba_hbm, out_hbm,
               ba_v, b_v, a_v, mr_v, pen_v, row_v, acc_v, acci_v, sem):
    is0 = (lax.axis_index("c") == 0) & (lax.axis_index("s") == 0)

    @pl.when(is0)
    def _():
        pltpu.sync_copy(ba_hbm, ba_v)
        lanes = lax.broadcasted_iota(jnp.int32, (16,), 0)

        for k in range(_MP // 16):
            b_v[pl.ds(k * 16, 16)] = lax.bitcast_convert_type(
                ba_v[pl.ds(k * 16, 16)], jnp.float32
            )
            a_v[pl.ds(k * 16, 16)] = ba_v[pl.ds(_MP + k * 16, 16)]
            mr_v[pl.ds(k * 16, 16)] = jnp.full((16,), -1, jnp.int32)

        zeros16 = jnp.zeros((16,), jnp.float32)

        @pl.loop(0, _NP // (16 * 14))
        def _(i):
            for u in range(14):
                pen_v[pl.ds((i * 14 + u) * 16, 16)] = zeros16

        def scan_row_best():
            acc_v[...] = row_v[pl.ds(0, 16)] + pen_v[pl.ds(0, 16)]
            acci_v[...] = lanes

            @pl.loop(0, (_NP // 16 - 1) // 5)
            def _(i):
                for u in range(5):
                    j = 1 + i * 5 + u
                    v = row_v[pl.ds(j * 16, 16)] + pen_v[pl.ds(j * 16, 16)]
                    cur = acc_v[...]
                    take = v > cur
                    acc_v[...] = jnp.where(take, v, cur)
                    acci_v[...] = jnp.where(take, lanes + j * 16, acci_v[...])

            mx = jnp.max(acc_v[...])
            arg = jnp.min(jnp.where(acc_v[...] == mx, acci_v[...], jnp.int32(_BIG)))
            return mx, arg

        def update_gt(g, newb, newa, alive_only):
            off = (g // 16) * 16
            lsel = lanes == (g % 16)
            if alive_only:
                lsel = lsel & (b_v[pl.ds(off, 16)] > -1.5)
            b_v[pl.ds(off, 16)] = jnp.where(lsel, newb, b_v[pl.ds(off, 16)])
            a_v[pl.ds(off, 16)] = jnp.where(lsel, newa, a_v[pl.ds(off, 16)])

        def rescan(c):
            pltpu.async_copy(iou_hbm.at[c], row_v, sem).wait()
            mx2, arg2 = scan_row_best()
            update_gt(c, mx2, arg2, False)

        def select():
            val = b_v[pl.ds(0, 16)]
            pk = a_v[pl.ds(0, 16)] * _MP + lanes
            for k in range(1, _MP // 16):
                v = b_v[pl.ds(k * 16, 16)]
                p = a_v[pl.ds(k * 16, 16)] * _MP + (lanes + k * 16)
                take = (v > val) | ((v == val) & (p < pk))
                val = jnp.where(take, v, val)
                pk = jnp.where(take, p, pk)
            mx = jnp.max(val)
            pkm = jnp.min(jnp.where(val == mx, pk, jnp.int32(_BIG)))
            r = pkm // _MP
            c = pkm % _MP
            po = (r // 16) * 16
            pr = jnp.max(jnp.where(lanes == (r % 16), pen_v[pl.ds(po, 16)], -1e30))
            return r, c, mx > 1e-12, pr < -2.5

        def commit(r, c):
            goff = (c // 16) * 16
            gsel = lanes == (c % 16)
            mr_v[pl.ds(goff, 16)] = jnp.where(gsel, r, mr_v[pl.ds(goff, 16)])
            b_v[pl.ds(goff, 16)] = jnp.where(
                gsel, jnp.float32(-2.0), b_v[pl.ds(goff, 16)]
            )
            po = (r // 16) * 16
            psel = lanes == (r % 16)
            pen_v[pl.ds(po, 16)] = jnp.where(
                psel, jnp.float32(-3.0), pen_v[pl.ds(po, 16)]
            )

        @pl.loop(0, _M)
        def _(_round):
            r1, c1, valid1, stale1 = select()

            @pl.when(valid1 & jnp.logical_not(stale1))
            def _():
                commit(r1, c1)

            @pl.when(valid1 & stale1)
            def _():
                rescan(c1)
                r2, c2, valid2, stale2 = select()

                @pl.when(valid2 & jnp.logical_not(stale2))
                def _():
                    commit(r2, c2)

                @pl.when(valid2 & stale2)
                def _():
                    rescan(c2)
                    r3, c3, valid3, stale3 = select()

                    @pl.when(valid3 & jnp.logical_not(stale3))
                    def _():
                        commit(r3, c3)

                    @pl.when(valid3 & stale3)
                    def _():
                        @pl.loop(0, _M)
                        def _(g):
                            pltpu.async_copy(iou_hbm.at[g], row_v, sem).wait()
                            mxg, argg = scan_row_best()
                            update_gt(g, mxg, argg, True)

                        r4, c4, valid4, _stale4 = select()

                        @pl.when(valid4)
                        def _():
                            commit(r4, c4)

        pltpu.sync_copy(mr_v, out_hbm)


def _tc_c_body(at_ref, gt_ref, gid_ref, mr_ref, mm_ref,
               cls_ref, box_ref, msk_ref):
    at = at_ref[...]
    gt = gt_ref[...]
    _, (ax1, ay1, ax2, ay2) = _iou_parts(at, gt)
    gx1, gy1, gx2, gy2 = gt[:, 0:1], gt[:, 1:2], gt[:, 2:3], gt[:, 3:4]

    arow = lax.broadcasted_iota(jnp.int32, (1, _N), 1)
    g_iota64 = lax.broadcasted_iota(jnp.int32, (_MP, _N), 0)
    mr = mr_ref[...]
    bip = jnp.max(
        jnp.where(mr == arow, g_iota64, jnp.int32(-1)), axis=0, keepdims=True
    )

    mm = mm_ref[...]
    matches = jnp.where(bip >= 0, bip.astype(jnp.float32), mm)
    pos = matches >= 0.0
    safe = jnp.clip(matches, 0.0, float(_M - 1)).astype(jnp.int32)

    g_iota = lax.broadcasted_iota(jnp.int32, (_M, _N), 0)
    onehot = g_iota == safe

    def gsel(col):
        return jnp.max(jnp.where(onehot, col, -1e30), axis=0, keepdims=True)

    gid = gid_ref[...]
    rid = gsel(gid)
    rx1 = gsel(gx1)
    ry1 = gsel(gy1)
    rx2 = gsel(gx2)
    ry2 = gsel(gy2)

    cls_ref[...] = jnp.where(pos, rid + 1.0, 0.0)

    gw = rx2 - rx1
    gh = ry2 - ry1
    gx = rx1 + gw * 0.5
    gy = ry1 + gh * 0.5
    aw = ax2 - ax1
    ah = ay2 - ay1
    axc = ax1 + aw * 0.5
    ayc = ay1 + ah * 0.5
    t0 = ((gx - axc) / (aw + 1e-12)) / _STDS[0]
    t1 = ((gy - ayc) / (ah + 1e-12)) / _STDS[1]
    t2 = jnp.log(jnp.maximum(gw / (aw + 1e-12), 1e-12)) / _STDS[2]
    t3 = jnp.log(jnp.maximum(gh / (ah + 1e-12), 1e-12)) / _STDS[3]
    codes = jnp.concatenate([t0, t1, t2, t3], axis=0)

    posf = pos.astype(jnp.float32)
    box_ref[...] = jnp.transpose(codes * posf)
    msk_ref[...] = jnp.transpose(jnp.broadcast_to(posf, (4, _N)))


def _run_sc_match(iou_p, ba):
    mesh = plsc.VectorSubcoreMesh(
        core_axis_name="c", subcore_axis_name="s", num_cores=2, num_subcores=16
    )
    cp = pltpu.CompilerParams()
    if "needs_layout_passes" in pltpu.CompilerParams.__dataclass_fields__:
        cp = dataclasses.replace(cp, needs_layout_passes=False)
    return pl.kernel(
        _sc_b_body,
        out_type=jax.ShapeDtypeStruct((_MP,), jnp.int32),
        mesh=mesh,
        scratch_types=[
            pltpu.VMEM((2 * _MP,), jnp.int32),
            pltpu.VMEM((_MP,), jnp.float32),
            pltpu.VMEM((_MP,), jnp.int32),
            pltpu.VMEM((_MP,), jnp.int32),
            pltpu.VMEM((_NP,), jnp.float32),
            pltpu.VMEM((_NP,), jnp.float32),
            pltpu.VMEM((16,), jnp.float32),
            pltpu.VMEM((16,), jnp.int32),
            pltpu.SemaphoreType.DMA,
        ],
        compiler_params=cp,
    )(iou_p, ba)


@jax.jit
def kernel(anchors, gt_boxes, gt_ids):
    anchors_t = anchors.T
    iou_p, ba, mm = pl.pallas_call(
        _tc_a_body,
        out_shape=(
            jax.ShapeDtypeStruct((_M, _NP), jnp.float32),
            jax.ShapeDtypeStruct((2 * _MP, 1), jnp.int32),
            jax.ShapeDtypeStruct((1, _N), jnp.float32),
        ),
    )(anchors_t, gt_boxes)

    mr = _run_sc_match(iou_p, ba.reshape(2 * _MP))

    cls, box, msk = pl.pallas_call(
        _tc_c_body,
        out_shape=(
            jax.ShapeDtypeStruct((1, _N), jnp.float32),
            jax.ShapeDtypeStruct((_N, 4), jnp.float32),
            jax.ShapeDtypeStruct((_N, 4), jnp.float32),
        ),
    )(anchors_t, gt_boxes, gt_ids, mr.reshape(_MP, 1), mm)

    return cls, box[None, :, :], msk[None, :, :]


# speedup vs baseline: 1.1588x; 1.0593x over previous
import dataclasses

import jax
import jax.numpy as jnp
from jax import lax
from jax.experimental import pallas as pl
from jax.experimental.pallas import tpu as pltpu
from jax.experimental.pallas import tpu_sc as plsc

_N = 8732
_NP = 8736
_M = 50
_MP = 64
_IOU_THRESH = 0.5
_STDS = (0.1, 0.1, 0.2, 0.2)
_BIG = 2**30


def _iou_parts(at, gt):
    cx, cy, w, h = at[0:1, :], at[1:2, :], at[2:3, :], at[3:4, :]
    ax1 = cx - w * 0.5
    ay1 = cy - h * 0.5
    ax2 = cx + w * 0.5
    ay2 = cy + h * 0.5
    gx1, gy1, gx2, gy2 = gt[:, 0:1], gt[:, 1:2], gt[:, 2:3], gt[:, 3:4]
    iw = jnp.maximum(jnp.minimum(ax2, gx2) - jnp.maximum(ax1, gx1), 0.0)
    ih = jnp.maximum(jnp.minimum(ay2, gy2) - jnp.maximum(ay1, gy1), 0.0)
    inter = iw * ih
    area_a = (ax2 - ax1) * (ay2 - ay1)
    area_g = (gx2 - gx1) * (gy2 - gy1)
    iou = inter / (area_a + area_g - inter + 1e-12)
    return iou, (ax1, ay1, ax2, ay2)


def _tc_a_body(at_ref, gt_ref, iou_ref, ba_ref, mm_ref):
    iou, _ = _iou_parts(at_ref[...], gt_ref[...])
    iou_ref[:, :_N] = iou
    iou_ref[:, _N:] = jnp.full((_M, _NP - _N), -1.0, jnp.float32)
    a_iota = lax.broadcasted_iota(jnp.int32, (_M, _N), 1)
    g_iota = lax.broadcasted_iota(jnp.int32, (_M, _N), 0)
    bmax = jnp.max(iou, axis=1, keepdims=True)
    amin = jnp.min(
        jnp.where(iou == bmax, a_iota, jnp.int32(_BIG)), axis=1, keepdims=True
    )
    neg2 = jnp.full((1, _MP - _M), -1073741824, jnp.int32)
    ba_ref[...] = jnp.concatenate(
        [
            lax.bitcast_convert_type(jnp.transpose(bmax), jnp.int32),
            neg2,
            jnp.transpose(amin),
            jnp.zeros((1, _MP - _M), jnp.int32),
        ],
        axis=1,
    )
    mm_max = jnp.max(iou, axis=0, keepdims=True)
    mm_arg = jnp.min(
        jnp.where(iou == mm_max, g_iota, jnp.int32(_BIG)), axis=0, keepdims=True
    )
    mm_ref[...] = jnp.where(mm_max >= _IOU_THRESH, mm_arg.astype(jnp.float32), -1.0)


def _sc_b_body(iou_hbm, ba_hbm, out_hbm,
               ba_v, b_v, a_v, mr_v, pen_v, row_v, acc_v, acci_v, sem):
    is0 = (lax.axis_index("c") == 0) & (lax.axis_index("s") == 0)

    @pl.when(is0)
    def _():
        pltpu.sync_copy(ba_hbm, ba_v)
        lanes = lax.broadcasted_iota(jnp.int32, (16,), 0)

        for k in range(_MP // 16):
            b_v[pl.ds(k * 16, 16)] = lax.bitcast_convert_type(
                ba_v[pl.ds(k * 16, 16)], jnp.float32
            )
            a_v[pl.ds(k * 16, 16)] = ba_v[pl.ds(_MP + k * 16, 16)]
            mr_v[pl.ds(k * 16, 16)] = jnp.full((16,), -1, jnp.int32)

        zeros16 = jnp.zeros((16,), jnp.float32)

        @pl.loop(0, _NP // (16 * 14))
        def _(i):
            for u in range(14):
                pen_v[pl.ds((i * 14 + u) * 16, 16)] = zeros16

        def scan_row_best():
            acc_v[...] = row_v[pl.ds(0, 16)] + pen_v[pl.ds(0, 16)]
            acci_v[...] = lanes

            @pl.loop(0, (_NP // 16 - 1) // 5)
            def _(i):
                for u in range(5):
                    j = 1 + i * 5 + u
                    v = row_v[pl.ds(j * 16, 16)] + pen_v[pl.ds(j * 16, 16)]
                    cur = acc_v[...]
                    take = v > cur
                    acc_v[...] = jnp.where(take, v, cur)
                    acci_v[...] = jnp.where(take, lanes + j * 16, acci_v[...])

            mx = jnp.max(acc_v[...])
            arg = jnp.min(jnp.where(acc_v[...] == mx, acci_v[...], jnp.int32(_BIG)))
            return mx, arg

        def update_gt(g, newb, newa, alive_only):
            off = (g // 16) * 16
            lsel = lanes == (g % 16)
            if alive_only:
                lsel = lsel & (b_v[pl.ds(off, 16)] > -1.5)
            b_v[pl.ds(off, 16)] = jnp.where(lsel, newb, b_v[pl.ds(off, 16)])
            a_v[pl.ds(off, 16)] = jnp.where(lsel, newa, a_v[pl.ds(off, 16)])

        def rescan(c):
            pltpu.async_copy(iou_hbm.at[c], row_v, sem).wait()
            mx2, arg2 = scan_row_best()
            update_gt(c, mx2, arg2, False)

        def select():
            val = b_v[pl.ds(0, 16)]
            pk = a_v[pl.ds(0, 16)] * _MP + lanes
            for k in range(1, _MP // 16):
                v = b_v[pl.ds(k * 16, 16)]
                p = a_v[pl.ds(k * 16, 16)] * _MP + (lanes + k * 16)
                take = (v > val) | ((v == val) & (p < pk))
                val = jnp.where(take, v, val)
                pk = jnp.where(take, p, pk)
            mx = jnp.max(val)
            pkm = jnp.min(jnp.where(val == mx, pk, jnp.int32(_BIG)))
            r = pkm // _MP
            c = pkm % _MP
            po = (r // 16) * 16
            pr = jnp.max(jnp.where(lanes == (r % 16), pen_v[pl.ds(po, 16)], -1e30))
            return r, c, mx > 1e-12, pr < -2.5

        def commit(r, c):
            goff = (c // 16) * 16
            gsel = lanes == (c % 16)
            mr_v[pl.ds(goff, 16)] = jnp.where(gsel, r, mr_v[pl.ds(goff, 16)])
            b_v[pl.ds(goff, 16)] = jnp.where(
                gsel, jnp.float32(-2.0), b_v[pl.ds(goff, 16)]
            )
            po = (r // 16) * 16
            psel = lanes == (r % 16)
            pen_v[pl.ds(po, 16)] = jnp.where(
                psel, jnp.float32(-3.0), pen_v[pl.ds(po, 16)]
            )

        @pl.loop(0, _M)
        def _(_round):
            r1, c1, valid1, stale1 = select()

            @pl.when(valid1 & jnp.logical_not(stale1))
            def _():
                commit(r1, c1)

            @pl.when(valid1 & stale1)
            def _():
                rescan(c1)
                r2, c2, valid2, stale2 = select()

                @pl.when(valid2 & jnp.logical_not(stale2))
                def _():
                    commit(r2, c2)

                @pl.when(valid2 & stale2)
                def _():
                    rescan(c2)
                    r3, c3, valid3, stale3 = select()

                    @pl.when(valid3 & jnp.logical_not(stale3))
                    def _():
                        commit(r3, c3)

                    @pl.when(valid3 & stale3)
                    def _():
                        @pl.loop(0, _M)
                        def _(g):
                            pltpu.async_copy(iou_hbm.at[g], row_v, sem).wait()
                            mxg, argg = scan_row_best()
                            update_gt(g, mxg, argg, True)

                        r4, c4, valid4, _stale4 = select()

                        @pl.when(valid4)
                        def _():
                            commit(r4, c4)

        pltpu.sync_copy(mr_v, out_hbm.at[0])


def _tc_c_body(at_ref, gt_ref, gid_ref, mr_ref, mm_ref,
               cls_ref, box_ref, msk_ref):
    at = at_ref[...]
    gt = gt_ref[...]
    _, (ax1, ay1, ax2, ay2) = _iou_parts(at, gt)
    gx1, gy1, gx2, gy2 = gt[:, 0:1], gt[:, 1:2], gt[:, 2:3], gt[:, 3:4]

    arow = lax.broadcasted_iota(jnp.int32, (1, _N), 1)
    g_iota64 = lax.broadcasted_iota(jnp.int32, (_MP, _N), 0)
    mr = jnp.transpose(mr_ref[...])
    bip = jnp.max(
        jnp.where(mr == arow, g_iota64, jnp.int32(-1)), axis=0, keepdims=True
    )

    mm = mm_ref[...]
    matches = jnp.where(bip >= 0, bip.astype(jnp.float32), mm)
    pos = matches >= 0.0
    safe = jnp.clip(matches, 0.0, float(_M - 1)).astype(jnp.int32)

    g_iota = lax.broadcasted_iota(jnp.int32, (_M, _N), 0)
    onehot = g_iota == safe

    def gsel(col):
        return jnp.max(jnp.where(onehot, col, -1e30), axis=0, keepdims=True)

    gid = gid_ref[...]
    rid = gsel(gid)
    rx1 = gsel(gx1)
    ry1 = gsel(gy1)
    rx2 = gsel(gx2)
    ry2 = gsel(gy2)

    cls_ref[...] = jnp.where(pos, rid + 1.0, 0.0)

    gw = rx2 - rx1
    gh = ry2 - ry1
    gx = rx1 + gw * 0.5
    gy = ry1 + gh * 0.5
    aw = ax2 - ax1
    ah = ay2 - ay1
    axc = ax1 + aw * 0.5
    ayc = ay1 + ah * 0.5
    t0 = ((gx - axc) / (aw + 1e-12)) / _STDS[0]
    t1 = ((gy - ayc) / (ah + 1e-12)) / _STDS[1]
    t2 = jnp.log(jnp.maximum(gw / (aw + 1e-12), 1e-12)) / _STDS[2]
    t3 = jnp.log(jnp.maximum(gh / (ah + 1e-12), 1e-12)) / _STDS[3]
    codes = jnp.concatenate([t0, t1, t2, t3], axis=0)

    posf = pos.astype(jnp.float32)
    box_ref[...] = jnp.transpose(codes * posf)
    msk_ref[...] = jnp.transpose(jnp.broadcast_to(posf, (4, _N)))


def _run_sc_match(iou_p, ba):
    mesh = plsc.VectorSubcoreMesh(
        core_axis_name="c", subcore_axis_name="s", num_cores=2, num_subcores=16
    )
    cp = pltpu.CompilerParams()
    if "needs_layout_passes" in pltpu.CompilerParams.__dataclass_fields__:
        cp = dataclasses.replace(cp, needs_layout_passes=False)
    return pl.kernel(
        _sc_b_body,
        out_type=jax.ShapeDtypeStruct((1, _MP), jnp.int32),
        mesh=mesh,
        scratch_types=[
            pltpu.VMEM((2 * _MP,), jnp.int32),
            pltpu.VMEM((_MP,), jnp.float32),
            pltpu.VMEM((_MP,), jnp.int32),
            pltpu.VMEM((_MP,), jnp.int32),
            pltpu.VMEM((_NP,), jnp.float32),
            pltpu.VMEM((_NP,), jnp.float32),
            pltpu.VMEM((16,), jnp.float32),
            pltpu.VMEM((16,), jnp.int32),
            pltpu.SemaphoreType.DMA,
        ],
        compiler_params=cp,
    )(iou_p, ba)


@jax.jit
def kernel(anchors, gt_boxes, gt_ids):
    anchors_t = anchors.T
    iou_p, ba, mm = pl.pallas_call(
        _tc_a_body,
        out_shape=(
            jax.ShapeDtypeStruct((_M, _NP), jnp.float32),
            jax.ShapeDtypeStruct((1, 2 * _MP), jnp.int32),
            jax.ShapeDtypeStruct((1, _N), jnp.float32),
        ),
    )(anchors_t, gt_boxes)

    mr = _run_sc_match(iou_p, ba.reshape(2 * _MP))

    cls, box, msk = pl.pallas_call(
        _tc_c_body,
        out_shape=(
            jax.ShapeDtypeStruct((1, _N), jnp.float32),
            jax.ShapeDtypeStruct((_N, 4), jnp.float32),
            jax.ShapeDtypeStruct((_N, 4), jnp.float32),
        ),
    )(anchors_t, gt_boxes, gt_ids, mr, mm)

    return cls, box[None, :, :], msk[None, :, :]
